# baseline (speedup 1.0000x reference)
import functools

import jax
import jax.numpy as jnp
from jax import lax
from jax.experimental import pallas as pl
from jax.experimental.pallas import tpu as pltpu
from jax.experimental.pallas import tpu_sc as plsc

B = 16384
D = 16
NC = 2
NS = 16
NW = NC * NS
BPW = B // NW

_mesh = plsc.VectorSubcoreMesh(core_axis_name="c", subcore_axis_name="s")


@functools.partial(
    pl.kernel,
    mesh=_mesh,
    compiler_params=pltpu.CompilerParams(use_tc_tiling_on_sc=False),
    out_type=(
        jax.ShapeDtypeStruct((D, B), jnp.float32),
        jax.ShapeDtypeStruct((D, B), jnp.float32),
    ),
    scratch_types=[
        pltpu.VMEM((BPW,), jnp.int32),
        pltpu.VMEM((BPW,), jnp.int32),
        pltpu.VMEM((D, BPW), jnp.float32),
        pltpu.VMEM((D, BPW), jnp.float32),
        pltpu.SemaphoreType.DMA,
    ],
)
def _gather_sc(uidx_hbm, iidx_hbm, utabT_hbm, itabT_hbm, uoutT_hbm, ioutT_hbm,
               uidx_v, iidx_v, urowsT_v, irowsT_v, sem):
    wid = lax.axis_index("s") * NC + lax.axis_index("c")
    base = wid * BPW
    pltpu.sync_copy(uidx_hbm.at[pl.ds(base, BPW)], uidx_v)
    pltpu.sync_copy(iidx_hbm.at[pl.ds(base, BPW)], iidx_v)
    copies = []
    for d in range(D):
        copies.append(
            pltpu.async_copy(utabT_hbm.at[d].at[uidx_v], urowsT_v.at[d], sem))
        copies.append(
            pltpu.async_copy(itabT_hbm.at[d].at[iidx_v], irowsT_v.at[d], sem))
    for c in copies:
        c.wait()
    pltpu.sync_copy(urowsT_v, uoutT_hbm.at[:, pl.ds(base, BPW)])
    pltpu.sync_copy(irowsT_v, ioutT_hbm.at[:, pl.ds(base, BPW)])


def _mlp_body(ue_ref, ie_ref, w1uT_ref, w1iT_ref, b1_ref, w2T_ref, b2_ref,
              w3T_ref, b3_ref, out_ref):
    x = (jnp.dot(w1uT_ref[...], ue_ref[...], preferred_element_type=jnp.float32)
         + jnp.dot(w1iT_ref[...], ie_ref[...], preferred_element_type=jnp.float32)
         + b1_ref[...])
    x = jnp.maximum(x, 0.0)
    x = jnp.maximum(
        jnp.dot(w2T_ref[...], x, preferred_element_type=jnp.float32) + b2_ref[...],
        0.0)
    x = jnp.dot(w3T_ref[...], x, preferred_element_type=jnp.float32) + b3_ref[...]
    out_ref[...] = jax.nn.sigmoid(x)


_COLS_BLK = 2048
_GRID = B // _COLS_BLK

_mlp_tc = pl.pallas_call(
    _mlp_body,
    grid=(_GRID,),
    in_specs=[
        pl.BlockSpec((D, _COLS_BLK), lambda i: (0, i)),
        pl.BlockSpec((D, _COLS_BLK), lambda i: (0, i)),
        pl.BlockSpec((16, D), lambda i: (0, 0)),
        pl.BlockSpec((16, D), lambda i: (0, 0)),
        pl.BlockSpec((16, 1), lambda i: (0, 0)),
        pl.BlockSpec((8, 16), lambda i: (0, 0)),
        pl.BlockSpec((8, 1), lambda i: (0, 0)),
        pl.BlockSpec((1, 8), lambda i: (0, 0)),
        pl.BlockSpec((1, 1), lambda i: (0, 0)),
    ],
    out_specs=pl.BlockSpec((1, _COLS_BLK), lambda i: (0, i)),
    out_shape=jax.ShapeDtypeStruct((1, B), jnp.float32),
)


def kernel(user_indices, item_indices, user_table, item_table,
           W1, b1, W2, b2, W3, b3):
    ueT, ieT = _gather_sc(user_indices, item_indices,
                          user_table.T, item_table.T)
    out = _mlp_tc(ueT, ieT, W1[:D].T, W1[D:].T, b1.reshape(16, 1),
                  W2.T, b2.reshape(8, 1), W3.T, b3.reshape(1, 1))
    return out[0]


# speedup vs baseline: 14.3978x; 14.3978x over previous
import functools

import jax
import jax.numpy as jnp
from jax import lax
from jax.experimental import pallas as pl
from jax.experimental.pallas import tpu as pltpu
from jax.experimental.pallas import tpu_sc as plsc

B = 16384
D = 16
V = 1000000
NC = 2
NS = 16
NW = NC * NS
BPW = B // NW
CH = 16
NCHUNK = BPW // CH

_mesh = plsc.VectorSubcoreMesh(core_axis_name="c", subcore_axis_name="s")


@functools.partial(
    pl.kernel,
    mesh=_mesh,
    compiler_params=pltpu.CompilerParams(use_tc_tiling_on_sc=True,
                                         needs_layout_passes=False),
    out_type=(
        jax.ShapeDtypeStruct((D, B), jnp.float32),
        jax.ShapeDtypeStruct((D, B), jnp.float32),
    ),
    scratch_types=[
        pltpu.VMEM((BPW,), jnp.int32),
        pltpu.VMEM((BPW,), jnp.int32),
        pltpu.VMEM((CH, D, 128), jnp.float32),
        pltpu.VMEM((D, BPW), jnp.float32),
        pltpu.VMEM((D, BPW), jnp.float32),
        pltpu.SemaphoreType.DMA,
    ],
)
def _gather_sc(uidx_hbm, iidx_hbm, utabT_hbm, itabT_hbm, uoutT_hbm, ioutT_hbm,
               uidx_v, iidx_v, blkbuf, urowsT_v, irowsT_v, sem):
    wid = lax.axis_index("s") * NC + lax.axis_index("c")
    base = wid * BPW
    pltpu.sync_copy(uidx_hbm.at[pl.ds(base, BPW)], uidx_v)
    pltpu.sync_copy(iidx_hbm.at[pl.ds(base, BPW)], iidx_v)
    lanes = lax.iota(jnp.int32, 16)

    def make_pass(idx_v, tab_hbm, rowsT_v):
        def body(c, carry):
            jv = idx_v[pl.ds(c * CH, CH)]
            blk = jax.lax.shift_right_logical(jv, 7)
            col = jax.lax.bitwise_and(jv, 127)
            for k in range(CH):
                off = pl.multiple_of(blk[k] * 128, 128)
                pltpu.make_async_copy(
                    tab_hbm.at[:, pl.ds(off, 128)], blkbuf.at[k], sem
                ).start()
            for k in range(CH):
                pltpu.make_async_copy(
                    tab_hbm.at[:, pl.ds(0, 128)], blkbuf.at[k], sem
                ).wait()
            for k in range(CH):
                v = plsc.load_gather(
                    blkbuf,
                    [jnp.full((16,), k, jnp.int32), lanes,
                     jnp.full((16,), col[k], jnp.int32)])
                plsc.store_scatter(
                    rowsT_v,
                    [lanes, jnp.full((16,), c * CH + k, jnp.int32)], v)
            return carry
        return body

    lax.fori_loop(0, NCHUNK, make_pass(uidx_v, utabT_hbm, urowsT_v), 0)
    lax.fori_loop(0, NCHUNK, make_pass(iidx_v, itabT_hbm, irowsT_v), 0)
    pltpu.sync_copy(urowsT_v, uoutT_hbm.at[:, pl.ds(base, BPW)])
    pltpu.sync_copy(irowsT_v, ioutT_hbm.at[:, pl.ds(base, BPW)])


def _mlp_body(ue_ref, ie_ref, w1uT_ref, w1iT_ref, b1_ref, w2T_ref, b2_ref,
              w3T_ref, b3_ref, out_ref):
    x = (jnp.dot(w1uT_ref[...], ue_ref[...], preferred_element_type=jnp.float32)
         + jnp.dot(w1iT_ref[...], ie_ref[...], preferred_element_type=jnp.float32)
         + b1_ref[...])
    x = jnp.maximum(x, 0.0)
    x = jnp.maximum(
        jnp.dot(w2T_ref[...], x, preferred_element_type=jnp.float32) + b2_ref[...],
        0.0)
    x = jnp.dot(w3T_ref[...], x, preferred_element_type=jnp.float32) + b3_ref[...]
    out_ref[...] = jax.nn.sigmoid(x)


_COLS_BLK = 2048
_GRID = B // _COLS_BLK

_mlp_tc = pl.pallas_call(
    _mlp_body,
    grid=(_GRID,),
    in_specs=[
        pl.BlockSpec((D, _COLS_BLK), lambda i: (0, i)),
        pl.BlockSpec((D, _COLS_BLK), lambda i: (0, i)),
        pl.BlockSpec((16, D), lambda i: (0, 0)),
        pl.BlockSpec((16, D), lambda i: (0, 0)),
        pl.BlockSpec((16, 1), lambda i: (0, 0)),
        pl.BlockSpec((8, 16), lambda i: (0, 0)),
        pl.BlockSpec((8, 1), lambda i: (0, 0)),
        pl.BlockSpec((1, 8), lambda i: (0, 0)),
        pl.BlockSpec((1, 1), lambda i: (0, 0)),
    ],
    out_specs=pl.BlockSpec((1, _COLS_BLK), lambda i: (0, i)),
    out_shape=jax.ShapeDtypeStruct((1, B), jnp.float32),
)


def kernel(user_indices, item_indices, user_table, item_table,
           W1, b1, W2, b2, W3, b3):
    ueT, ieT = _gather_sc(user_indices, item_indices,
                          user_table.T, item_table.T)
    out = _mlp_tc(ueT, ieT, W1[:D].T, W1[D:].T, b1.reshape(16, 1),
                  W2.T, b2.reshape(8, 1), W3.T, b3.reshape(1, 1))
    return out[0]


# speedup vs baseline: 20.8474x; 1.4480x over previous
import functools

import jax
import jax.numpy as jnp
from jax import lax
from jax.experimental import pallas as pl
from jax.experimental.pallas import tpu as pltpu
from jax.experimental.pallas import tpu_sc as plsc

B = 16384
D = 16
V = 1000000
NC = 2
NS = 16
NW = NC * NS
BPW = B // NW
CH = 16
NCHUNK = BPW // CH

_mesh = plsc.VectorSubcoreMesh(core_axis_name="c", subcore_axis_name="s")


@functools.partial(
    pl.kernel,
    mesh=_mesh,
    compiler_params=pltpu.CompilerParams(use_tc_tiling_on_sc=True,
                                         needs_layout_passes=False),
    out_type=(
        jax.ShapeDtypeStruct((D, B), jnp.float32),
        jax.ShapeDtypeStruct((D, B), jnp.float32),
    ),
    scratch_types=[
        pltpu.VMEM((BPW,), jnp.int32),
        pltpu.VMEM((BPW,), jnp.int32),
        pltpu.VMEM((D, CH * 128), jnp.float32),
        pltpu.VMEM((D, CH * 128), jnp.float32),
        pltpu.VMEM((D, BPW), jnp.float32),
        pltpu.VMEM((D, BPW), jnp.float32),
        pltpu.SemaphoreType.DMA,
    ],
)
def _gather_sc(uidx_hbm, iidx_hbm, utabT_hbm, itabT_hbm, uoutT_hbm, ioutT_hbm,
               uidx_v, iidx_v, blkbuf0, blkbuf1, urowsT_v, irowsT_v, sem):
    wid = lax.axis_index("s") * NC + lax.axis_index("c")
    base = wid * BPW
    pltpu.sync_copy(uidx_hbm.at[pl.ds(base, BPW)], uidx_v)
    pltpu.sync_copy(iidx_hbm.at[pl.ds(base, BPW)], iidx_v)
    lanes = lax.iota(jnp.int32, 16)

    def run_pass(idx_v, tab_hbm, rowsT_v):
        def fire(c, buf):
            jv = idx_v[pl.ds(c * CH, CH)]
            blk = jax.lax.shift_right_logical(jv, 7)
            for k in range(CH):
                off = pl.multiple_of(blk[k] * 128, 128)
                pltpu.make_async_copy(
                    tab_hbm.at[:, pl.ds(off, 128)],
                    buf.at[:, pl.ds(k * 128, 128)], sem
                ).start()

        def wait_chunk(buf):
            pltpu.make_async_copy(
                tab_hbm.at[:, pl.ds(0, CH * 128)], buf, sem).wait()

        def extract(c, buf):
            jv = idx_v[pl.ds(c * CH, CH)]
            col = jax.lax.bitwise_and(jv, 127)
            for k in range(CH):
                v = plsc.load_gather(
                    buf,
                    [lanes, jnp.full((16,), k * 128, jnp.int32) + col[k]])
                plsc.store_scatter(
                    rowsT_v,
                    [lanes, jnp.full((16,), c * CH + k, jnp.int32)], v)

        fire(0, blkbuf0)

        def body(p, carry):
            c0 = p * 2
            fire(c0 + 1, blkbuf1)
            wait_chunk(blkbuf0)
            extract(c0, blkbuf0)

            @pl.when(p < NCHUNK // 2 - 1)
            def _():
                fire(c0 + 2, blkbuf0)

            wait_chunk(blkbuf1)
            extract(c0 + 1, blkbuf1)
            return carry

        lax.fori_loop(0, NCHUNK // 2, body, 0)

    run_pass(uidx_v, utabT_hbm, urowsT_v)
    run_pass(iidx_v, itabT_hbm, irowsT_v)
    pltpu.sync_copy(urowsT_v, uoutT_hbm.at[:, pl.ds(base, BPW)])
    pltpu.sync_copy(irowsT_v, ioutT_hbm.at[:, pl.ds(base, BPW)])


def _mlp_body(ue_ref, ie_ref, w1uT_ref, w1iT_ref, b1_ref, w2T_ref, b2_ref,
              w3T_ref, b3_ref, out_ref):
    x = (jnp.dot(w1uT_ref[...], ue_ref[...], preferred_element_type=jnp.float32)
         + jnp.dot(w1iT_ref[...], ie_ref[...], preferred_element_type=jnp.float32)
         + b1_ref[...])
    x = jnp.maximum(x, 0.0)
    x = jnp.maximum(
        jnp.dot(w2T_ref[...], x, preferred_element_type=jnp.float32) + b2_ref[...],
        0.0)
    x = jnp.dot(w3T_ref[...], x, preferred_element_type=jnp.float32) + b3_ref[...]
    out_ref[...] = jax.nn.sigmoid(x)


_COLS_BLK = 2048
_GRID = B // _COLS_BLK

_mlp_tc = pl.pallas_call(
    _mlp_body,
    grid=(_GRID,),
    in_specs=[
        pl.BlockSpec((D, _COLS_BLK), lambda i: (0, i)),
        pl.BlockSpec((D, _COLS_BLK), lambda i: (0, i)),
        pl.BlockSpec((16, D), lambda i: (0, 0)),
        pl.BlockSpec((16, D), lambda i: (0, 0)),
        pl.BlockSpec((16, 1), lambda i: (0, 0)),
        pl.BlockSpec((8, 16), lambda i: (0, 0)),
        pl.BlockSpec((8, 1), lambda i: (0, 0)),
        pl.BlockSpec((1, 8), lambda i: (0, 0)),
        pl.BlockSpec((1, 1), lambda i: (0, 0)),
    ],
    out_specs=pl.BlockSpec((1, _COLS_BLK), lambda i: (0, i)),
    out_shape=jax.ShapeDtypeStruct((1, B), jnp.float32),
)


def kernel(user_indices, item_indices, user_table, item_table,
           W1, b1, W2, b2, W3, b3):
    ueT, ieT = _gather_sc(user_indices, item_indices,
                          user_table.T, item_table.T)
    out = _mlp_tc(ueT, ieT, W1[:D].T, W1[D:].T, b1.reshape(16, 1),
                  W2.T, b2.reshape(8, 1), W3.T, b3.reshape(1, 1))
    return out[0]


# speedup vs baseline: 21.5549x; 1.0339x over previous
import functools

import jax
import jax.numpy as jnp
from jax import lax
from jax.experimental import pallas as pl
from jax.experimental.pallas import tpu as pltpu
from jax.experimental.pallas import tpu_sc as plsc

B = 16384
D = 16
V = 1000000
NC = 2
NS = 16
NW = NC * NS
BPW = B // NW
CH = 16
NCHUNK = BPW // CH

_mesh = plsc.VectorSubcoreMesh(core_axis_name="c", subcore_axis_name="s")


@functools.partial(
    pl.kernel,
    mesh=_mesh,
    compiler_params=pltpu.CompilerParams(use_tc_tiling_on_sc=True,
                                         needs_layout_passes=False),
    out_type=(
        jax.ShapeDtypeStruct((D, B), jnp.float32),
        jax.ShapeDtypeStruct((D, B), jnp.float32),
    ),
    scratch_types=[
        pltpu.VMEM((BPW,), jnp.int32),
        pltpu.VMEM((BPW,), jnp.int32),
        pltpu.VMEM((D, CH * 128), jnp.float32),
        pltpu.VMEM((D, CH * 128), jnp.float32),
        pltpu.VMEM((D, BPW), jnp.float32),
        pltpu.VMEM((D, BPW), jnp.float32),
        pltpu.SemaphoreType.DMA,
    ],
)
def _gather_sc(uidx_hbm, iidx_hbm, utabT_hbm, itabT_hbm, uoutT_hbm, ioutT_hbm,
               uidx_v, iidx_v, blkbuf0, blkbuf1, urowsT_v, irowsT_v, sem):
    wid = lax.axis_index("s") * NC + lax.axis_index("c")
    base = wid * BPW
    pltpu.sync_copy(uidx_hbm.at[pl.ds(base, BPW)], uidx_v)
    pltpu.sync_copy(iidx_hbm.at[pl.ds(base, BPW)], iidx_v)
    lanes = lax.iota(jnp.int32, 16)

    def run_pass(idx_v, tab_hbm, rowsT_v):
        def fire(c, buf):
            jv = idx_v[pl.ds(c * CH, CH)]
            blk = jax.lax.shift_right_logical(jv, 7)
            for k in range(CH):
                off = pl.multiple_of(blk[k] * 128, 128)
                pltpu.make_async_copy(
                    tab_hbm.at[:, pl.ds(off, 128)],
                    buf.at[:, pl.ds(k * 128, 128)], sem
                ).start()

        def wait_chunk(buf):
            pltpu.make_async_copy(
                tab_hbm.at[:, pl.ds(0, CH * 128)], buf, sem).wait()

        def extract(c, buf):
            jv = idx_v[pl.ds(c * CH, CH)]
            col = jax.lax.bitwise_and(jv, 127)
            for k in range(CH):
                v = plsc.load_gather(
                    buf,
                    [lanes, jnp.full((16,), k * 128, jnp.int32) + col[k]])
                plsc.store_scatter(
                    rowsT_v,
                    [lanes, jnp.full((16,), c * CH + k, jnp.int32)], v)

        fire(0, blkbuf0)

        def body(p, carry):
            c0 = p * 2
            fire(c0 + 1, blkbuf1)
            wait_chunk(blkbuf0)
            extract(c0, blkbuf0)

            @pl.when(p < NCHUNK // 2 - 1)
            def _():
                fire(c0 + 2, blkbuf0)

            wait_chunk(blkbuf1)
            extract(c0 + 1, blkbuf1)
            return carry

        lax.fori_loop(0, NCHUNK // 2, body, 0)

    run_pass(uidx_v, utabT_hbm, urowsT_v)
    run_pass(iidx_v, itabT_hbm, irowsT_v)
    pltpu.sync_copy(urowsT_v, uoutT_hbm.at[:, pl.ds(base, BPW)])
    pltpu.sync_copy(irowsT_v, ioutT_hbm.at[:, pl.ds(base, BPW)])


def _mlp_body(ue_ref, ie_ref, w1uT_ref, w1iT_ref, b1_ref, w2T_ref, b2_ref,
              w3T_ref, b3_ref, out_ref):
    x = (jnp.dot(w1uT_ref[...], ue_ref[...], preferred_element_type=jnp.float32)
         + jnp.dot(w1iT_ref[...], ie_ref[...], preferred_element_type=jnp.float32)
         + b1_ref[...])
    x = jnp.maximum(x, 0.0)
    x = jnp.maximum(
        jnp.dot(w2T_ref[...], x, preferred_element_type=jnp.float32) + b2_ref[...],
        0.0)
    x = jnp.dot(w3T_ref[...], x, preferred_element_type=jnp.float32) + b3_ref[...]
    out_ref[...] = jax.nn.sigmoid(x)


_COLS_BLK = 4096
_GRID = B // _COLS_BLK

_mlp_tc = pl.pallas_call(
    _mlp_body,
    grid=(_GRID,),
    in_specs=[
        pl.BlockSpec((D, _COLS_BLK), lambda i: (0, i)),
        pl.BlockSpec((D, _COLS_BLK), lambda i: (0, i)),
        pl.BlockSpec((16, D), lambda i: (0, 0)),
        pl.BlockSpec((16, D), lambda i: (0, 0)),
        pl.BlockSpec((16, 1), lambda i: (0, 0)),
        pl.BlockSpec((8, 16), lambda i: (0, 0)),
        pl.BlockSpec((8, 1), lambda i: (0, 0)),
        pl.BlockSpec((1, 8), lambda i: (0, 0)),
        pl.BlockSpec((1, 1), lambda i: (0, 0)),
    ],
    out_specs=pl.BlockSpec((1, _COLS_BLK), lambda i: (0, i)),
    out_shape=jax.ShapeDtypeStruct((1, B), jnp.float32),
)


def kernel(user_indices, item_indices, user_table, item_table,
           W1, b1, W2, b2, W3, b3):
    ueT, ieT = _gather_sc(user_indices, item_indices,
                          user_table.T, item_table.T)
    out = _mlp_tc(ueT, ieT, W1[:D].T, W1[D:].T, b1.reshape(16, 1),
                  W2.T, b2.reshape(8, 1), W3.T, b3.reshape(1, 1))
    return out[0]


# speedup vs baseline: 21.6749x; 1.0056x over previous
import functools

import jax
import jax.numpy as jnp
from jax import lax
from jax.experimental import pallas as pl
from jax.experimental.pallas import tpu as pltpu
from jax.experimental.pallas import tpu_sc as plsc

B = 16384
D = 16
V = 1000000
NC = 2
NS = 16
NW = NC * NS
BPW = B // NW
CH = 16
NCHUNK = BPW // CH

_mesh = plsc.VectorSubcoreMesh(core_axis_name="c", subcore_axis_name="s")


@functools.partial(
    pl.kernel,
    mesh=_mesh,
    compiler_params=pltpu.CompilerParams(use_tc_tiling_on_sc=True,
                                         needs_layout_passes=False),
    out_type=(
        jax.ShapeDtypeStruct((D, B), jnp.float32),
        jax.ShapeDtypeStruct((D, B), jnp.float32),
    ),
    scratch_types=[
        pltpu.VMEM((BPW,), jnp.int32),
        pltpu.VMEM((BPW,), jnp.int32),
        pltpu.VMEM((D, CH * 128), jnp.float32),
        pltpu.VMEM((D, CH * 128), jnp.float32),
        pltpu.VMEM((D, BPW), jnp.float32),
        pltpu.VMEM((D, BPW), jnp.float32),
        pltpu.SemaphoreType.DMA,
    ],
)
def _gather_sc(uidx_hbm, iidx_hbm, utabT_hbm, itabT_hbm, uoutT_hbm, ioutT_hbm,
               uidx_v, iidx_v, blkbuf0, blkbuf1, urowsT_v, irowsT_v, sem):
    wid = lax.axis_index("s") * NC + lax.axis_index("c")
    base = wid * BPW
    pltpu.sync_copy(uidx_hbm.at[pl.ds(base, BPW)], uidx_v)
    pltpu.sync_copy(iidx_hbm.at[pl.ds(base, BPW)], iidx_v)
    lanes = lax.iota(jnp.int32, 16)

    def run_pass(idx_v, tab_hbm, rowsT_v):
        def fire(c, buf):
            jv = idx_v[pl.ds(c * CH, CH)]
            blk = jax.lax.shift_right_logical(jv, 7)
            for k in range(CH):
                off = pl.multiple_of(blk[k] * 128, 128)
                pltpu.make_async_copy(
                    tab_hbm.at[:, pl.ds(off, 128)],
                    buf.at[:, pl.ds(k * 128, 128)], sem
                ).start()

        def wait_chunk(buf):
            pltpu.make_async_copy(
                tab_hbm.at[:, pl.ds(0, CH * 128)], buf, sem).wait()

        def extract(c, buf):
            jv = idx_v[pl.ds(c * CH, CH)]
            col = jax.lax.bitwise_and(jv, 127)
            for k in range(CH):
                v = plsc.load_gather(
                    buf,
                    [lanes, jnp.full((16,), k * 128, jnp.int32) + col[k]])
                plsc.store_scatter(
                    rowsT_v,
                    [lanes, jnp.full((16,), c * CH + k, jnp.int32)], v)

        fire(0, blkbuf0)

        def body(p, carry):
            c0 = p * 2
            fire(c0 + 1, blkbuf1)
            wait_chunk(blkbuf0)
            extract(c0, blkbuf0)

            @pl.when(p < NCHUNK // 2 - 1)
            def _():
                fire(c0 + 2, blkbuf0)

            wait_chunk(blkbuf1)
            extract(c0 + 1, blkbuf1)
            return carry

        lax.fori_loop(0, NCHUNK // 2, body, 0)

    run_pass(uidx_v, utabT_hbm, urowsT_v)
    run_pass(iidx_v, itabT_hbm, irowsT_v)
    pltpu.sync_copy(urowsT_v, uoutT_hbm.at[:, pl.ds(base, BPW)])
    pltpu.sync_copy(irowsT_v, ioutT_hbm.at[:, pl.ds(base, BPW)])


def _mlp_body(ue_ref, ie_ref, w1uT_ref, w1iT_ref, b1_ref, w2T_ref, b2_ref,
              w3T_ref, b3_ref, out_ref):
    x = (jnp.dot(w1uT_ref[...], ue_ref[...], preferred_element_type=jnp.float32)
         + jnp.dot(w1iT_ref[...], ie_ref[...], preferred_element_type=jnp.float32)
         + b1_ref[...])
    x = jnp.maximum(x, 0.0)
    x = jnp.maximum(
        jnp.dot(w2T_ref[...], x, preferred_element_type=jnp.float32) + b2_ref[...],
        0.0)
    x = jnp.dot(w3T_ref[...], x, preferred_element_type=jnp.float32) + b3_ref[...]
    out_ref[...] = jax.nn.sigmoid(x)


_COLS_BLK = 16384
_GRID = B // _COLS_BLK

_mlp_tc = pl.pallas_call(
    _mlp_body,
    grid=(_GRID,),
    in_specs=[
        pl.BlockSpec((D, _COLS_BLK), lambda i: (0, i)),
        pl.BlockSpec((D, _COLS_BLK), lambda i: (0, i)),
        pl.BlockSpec((16, D), lambda i: (0, 0)),
        pl.BlockSpec((16, D), lambda i: (0, 0)),
        pl.BlockSpec((16, 1), lambda i: (0, 0)),
        pl.BlockSpec((8, 16), lambda i: (0, 0)),
        pl.BlockSpec((8, 1), lambda i: (0, 0)),
        pl.BlockSpec((1, 8), lambda i: (0, 0)),
        pl.BlockSpec((1, 1), lambda i: (0, 0)),
    ],
    out_specs=pl.BlockSpec((1, _COLS_BLK), lambda i: (0, i)),
    out_shape=jax.ShapeDtypeStruct((1, B), jnp.float32),
)


def kernel(user_indices, item_indices, user_table, item_table,
           W1, b1, W2, b2, W3, b3):
    ueT, ieT = _gather_sc(user_indices, item_indices,
                          user_table.T, item_table.T)
    out = _mlp_tc(ueT, ieT, W1[:D].T, W1[D:].T, b1.reshape(16, 1),
                  W2.T, b2.reshape(8, 1), W3.T, b3.reshape(1, 1))
    return out[0]
